# baseline (speedup 1.0000x reference)
import functools

import jax
import jax.numpy as jnp
from jax import lax
from jax.experimental import pallas as pl
from jax.experimental.pallas import tpu as pltpu
from jax.experimental.pallas import tpu_sc as plsc

N = 10000
NP = 10240
E = 160000
F = 256
G = 128
NG = 100
NCLS = 10
BN = 2048
BE = 2000
NBE = E // BE

NC = 2
NS = 16
NW = NC * NS
EPW = E // NW
ECH = 40
NBUF = 5
EGRP = EPW // (ECH * NBUF)
NPW = NP // NW
NCH = 80
NITER = NPW // NCH

_f32 = jnp.float32
_i32 = jnp.int32
_u32 = jnp.uint32
FH = F // 2


def _pack_bf16(x):
    u = lax.bitcast_convert_type(x, _u32)
    r = (u + 0x7FFF + ((u >> 16) & 1)) >> 16
    return r[:, :FH] | (r[:, FH:] << 16)


def _unpack_bf16(r):
    lo = lax.bitcast_convert_type(r << 16, _f32)
    hi = lax.bitcast_convert_type(r & _u32(0xFFFF0000), _f32)
    return lo, hi


def _g(x, m):
    x = jnp.where(jnp.abs(x) < jnp.inf, x, 0.0)
    return jnp.maximum(x, 0.0) * m


def _selu(x):
    alpha = 1.6732632423543772848170429916717
    scale = 1.0507009873554804934193349852946
    safe = jnp.minimum(x, 0.0)
    return scale * jnp.where(x > 0, x, alpha * (jnp.exp(safe) - 1.0))


def _stats_body(apply_g, x_ref, b_ref, m_ref, s1_ref, s2_ref, cnt_ref):
    pid = pl.program_id(0)

    @pl.when(pid == 0)
    def _():
        s1_ref[...] = jnp.zeros_like(s1_ref)
        s2_ref[...] = jnp.zeros_like(s2_ref)
        cnt_ref[...] = jnp.zeros_like(cnt_ref)

    if apply_g:
        lo, hi = _unpack_bf16(x_ref[...])
        x = _g(jnp.concatenate([lo, hi], axis=1), m_ref[...])
    else:
        x = x_ref[...]
    oh = (lax.broadcasted_iota(_i32, (BN, G), 1) == b_ref[...]).astype(_f32)
    dn = (((0,), (0,)), ((), ()))
    s1_ref[...] += lax.dot_general(oh, x, dn, preferred_element_type=_f32)
    s2_ref[...] += lax.dot_general(oh, x * x, dn, preferred_element_type=_f32)
    cnt_ref[...] += lax.dot_general(oh, jnp.ones((BN, G), _f32), dn,
                                    preferred_element_type=_f32)


def _stats(x, batch2d, mask2d, apply_g):
    grid = NP // BN
    return pl.pallas_call(
        functools.partial(_stats_body, apply_g),
        grid=(grid,),
        in_specs=[
            pl.BlockSpec((BN, FH if apply_g else F), lambda g: (g, 0)),
            pl.BlockSpec((BN, 1), lambda g: (g, 0)),
            pl.BlockSpec((BN, 1), lambda g: (g, 0)),
        ],
        out_specs=[
            pl.BlockSpec((G, F), lambda g: (0, 0)),
            pl.BlockSpec((G, F), lambda g: (0, 0)),
            pl.BlockSpec((G, G), lambda g: (0, 0)),
        ],
        out_shape=[
            jax.ShapeDtypeStruct((G, F), _f32),
            jax.ShapeDtypeStruct((G, F), _f32),
            jax.ShapeDtypeStruct((G, G), _f32),
        ],
    )(x, batch2d, mask2d)


def _pq_body(apply_g, x_ref, b_ref, m_ref, s1_ref, s2_ref, cnt_ref,
             w_ref, bb_ref, ms_ref, w1d_ref, w1b_ref, p_ref, q_ref):
    cnt = jnp.maximum(cnt_ref[:, 0:1], 1.0)
    mean = s1_ref[...] / cnt
    m2 = s2_ref[...] / cnt
    ms = ms_ref[...]
    var = m2 - mean * mean * ms * (2.0 - ms)
    scale = w_ref[...] * lax.rsqrt(var + 1e-5)
    shift = bb_ref[...] - scale * ms * mean
    if apply_g:
        lo, hi = _unpack_bf16(x_ref[...])
        x = _g(jnp.concatenate([lo, hi], axis=1), m_ref[...])
    else:
        x = x_ref[...]
    oh = (lax.broadcasted_iota(_i32, (BN, G), 1) == b_ref[...]).astype(_f32)
    xn = x * jnp.dot(oh, scale, preferred_element_type=_f32) \
        + jnp.dot(oh, shift, preferred_element_type=_f32)
    p_ref[...] = _pack_bf16(jnp.dot(xn, w1d_ref[...],
                                    preferred_element_type=_f32))
    q_ref[...] = _pack_bf16(jnp.dot(xn, w1b_ref[...],
                                    preferred_element_type=_f32))


def _apply_pq(x, batch2d, mask2d, s1, s2, cnt, w, bb, ms, w1d, w1b, apply_g):
    grid = NP // BN
    full = lambda g: (0, 0)
    blk = lambda g: (g, 0)
    return pl.pallas_call(
        functools.partial(_pq_body, apply_g),
        grid=(grid,),
        in_specs=[
            pl.BlockSpec((BN, FH if apply_g else F), blk),
            pl.BlockSpec((BN, 1), blk),
            pl.BlockSpec((BN, 1), blk),
            pl.BlockSpec((G, F), full),
            pl.BlockSpec((G, F), full),
            pl.BlockSpec((G, G), full),
            pl.BlockSpec((1, F), full),
            pl.BlockSpec((1, F), full),
            pl.BlockSpec((1, F), full),
            pl.BlockSpec((F, F), full),
            pl.BlockSpec((F, F), full),
        ],
        out_specs=[
            pl.BlockSpec((BN, FH), blk),
            pl.BlockSpec((BN, FH), blk),
        ],
        out_shape=[
            jax.ShapeDtypeStruct((NP, FH), _u32),
            jax.ShapeDtypeStruct((NP, FH), _u32),
        ],
    )(x, batch2d, mask2d, s1, s2, cnt, w, bb, ms, w1d, w1b)


def _edge_gather_body(p_hbm, q_hbm, d_hbm, s_hbm, rp_hbm, rq_hbm, *scr):
    wid = lax.axis_index("s") * NC + lax.axis_index("c")
    base = wid * EPW
    bufs = [scr[5 * b:5 * b + 5] for b in range(NBUF)]

    def group(i, carry):
        g0 = i * NBUF
        for b, (di, si, pr, qr, sem) in enumerate(bufs):
            off = base + (g0 + b) * ECH

            @pl.when(i > 0)
            def _(pr=pr, qr=qr, off=off, sem=sem):
                pltpu.make_async_copy(pr, rp_hbm.at[pl.ds(off, ECH)], sem).wait()
                pltpu.make_async_copy(qr, rq_hbm.at[pl.ds(off, ECH)], sem).wait()

            pltpu.sync_copy(d_hbm.at[pl.ds(off, ECH)], di)
            pltpu.sync_copy(s_hbm.at[pl.ds(off, ECH)], si)
            pltpu.async_copy(p_hbm.at[di], pr, sem)
            pltpu.async_copy(q_hbm.at[si], qr, sem)
        for b, (di, si, pr, qr, sem) in enumerate(bufs):
            off = base + (g0 + b) * ECH
            pltpu.make_async_copy(p_hbm.at[di], pr, sem).wait()
            pltpu.make_async_copy(q_hbm.at[si], qr, sem).wait()
            pltpu.async_copy(pr, rp_hbm.at[pl.ds(off, ECH)], sem)
            pltpu.async_copy(qr, rq_hbm.at[pl.ds(off, ECH)], sem)
        return carry

    lax.fori_loop(0, EGRP, group, 0)
    for di, si, pr, qr, sem in bufs:
        pltpu.make_async_copy(pr, rp_hbm.at[pl.ds(base, ECH)], sem).wait()
        pltpu.make_async_copy(qr, rq_hbm.at[pl.ds(base, ECH)], sem).wait()


def _edge_gather(p, q, dsts, srcs):
    mesh = plsc.VectorSubcoreMesh(core_axis_name="c", subcore_axis_name="s")
    scratch = []
    for _ in range(NBUF):
        scratch += [
            pltpu.VMEM((ECH,), _i32),
            pltpu.VMEM((ECH,), _i32),
            pltpu.VMEM((ECH, FH), _u32),
            pltpu.VMEM((ECH, FH), _u32),
            pltpu.SemaphoreType.DMA,
        ]
    fn = pl.kernel(
        _edge_gather_body,
        out_type=[
            jax.ShapeDtypeStruct((E, FH), _u32),
            jax.ShapeDtypeStruct((E, FH), _u32),
        ],
        mesh=mesh,
        scratch_types=scratch,
    )
    return fn(p, q, dsts, srcs)


def _mlp_segmax_body(rp_ref, rq_ref, d_ref, b1_ref, b2_ref, b3_ref,
                     w2_ref, w3_ref, out_ref, cd_ref, cv_ref):
    pid = pl.program_id(0)

    @pl.when(pid == 0)
    def _():
        cd_ref[...] = jnp.full(cd_ref.shape, -1, _i32)
        cv_ref[...] = jnp.full(cv_ref.shape, -jnp.inf, _f32)

    plo, phi = _unpack_bf16(rp_ref[...])
    qlo, qhi = _unpack_bf16(rq_ref[...])
    b1 = b1_ref[...]
    h = _selu(jnp.concatenate(
        [plo + qlo + b1[:, :FH], phi + qhi + b1[:, FH:]], axis=1))
    h = _selu(jnp.dot(h.astype(jnp.bfloat16), w2_ref[...],
                      preferred_element_type=_f32) + b2_ref[...])
    h = jnp.dot(h.astype(jnp.bfloat16), w3_ref[...],
                preferred_element_type=_f32) + b3_ref[...]
    d = d_ref[...]
    s = 1
    while s < BE:
        hs = jnp.concatenate([h[s:], jnp.zeros((s, F), _f32)], axis=0)
        ds = jnp.concatenate([d[s:], jnp.full((s, 1), -1, _i32)], axis=0)
        h = jnp.where(ds == d, jnp.maximum(h, hs), h)
        s *= 2
    cd = cd_ref[0:1, 0:1]
    cv = cv_ref[0:1, :]
    h = jnp.where(d == cd, jnp.maximum(h, cv), h)
    out_ref[...] = _pack_bf16(h)
    cd_ref[0:1, 0:1] = d[0:1, :]
    cv_ref[0:1, :] = h[0:1, :]


def _mlp_segmax(rp, rq, dsts2d, b1, b2, b3, w2, w3):
    desc = lambda g: (NBE - 1 - g, 0)
    full = lambda g: (0, 0)
    return pl.pallas_call(
        _mlp_segmax_body,
        grid=(NBE,),
        in_specs=[
            pl.BlockSpec((BE, FH), desc),
            pl.BlockSpec((BE, FH), desc),
            pl.BlockSpec((BE, 1), desc),
            pl.BlockSpec((1, F), full),
            pl.BlockSpec((1, F), full),
            pl.BlockSpec((1, F), full),
            pl.BlockSpec((F, F), full),
            pl.BlockSpec((F, F), full),
        ],
        out_specs=pl.BlockSpec((BE, FH), desc),
        out_shape=jax.ShapeDtypeStruct((E, FH), _u32),
        scratch_shapes=[
            pltpu.VMEM((8, 128), _i32),
            pltpu.VMEM((8, F), _f32),
        ],
    )(rp, rq, dsts2d, b1, b2, b3, w2, w3)


def _node_gather_body(s_hbm, idx_hbm, out_hbm, ix_v, rows_v, sem):
    wid = lax.axis_index("s") * NC + lax.axis_index("c")
    base = wid * NPW

    def step(j, carry):
        off = base + j * NCH
        pltpu.sync_copy(idx_hbm.at[pl.ds(off, NCH)], ix_v)
        pltpu.async_copy(s_hbm.at[ix_v], rows_v, sem).wait()
        pltpu.sync_copy(rows_v, out_hbm.at[pl.ds(off, NCH)])
        return carry

    lax.fori_loop(0, NITER, step, 0)


def _node_gather(seg, row_idx):
    mesh = plsc.VectorSubcoreMesh(core_axis_name="c", subcore_axis_name="s")
    fn = pl.kernel(
        _node_gather_body,
        out_type=jax.ShapeDtypeStruct((NP, FH), _u32),
        mesh=mesh,
        scratch_types=[
            pltpu.VMEM((NCH,), _i32),
            pltpu.VMEM((NCH, FH), _u32),
            pltpu.SemaphoreType.DMA,
        ],
    )
    return fn(seg, row_idx)


def _final_body(s1_ref, cnt_ref, wd_ref, bd_ref, wo_ref, bo_ref, out_ref):
    cnt = jnp.maximum(cnt_ref[:, 0:1], 1.0)
    pooled = s1_ref[...] / cnt
    dd = jnp.maximum(
        jnp.dot(pooled, wd_ref[...], preferred_element_type=_f32)
        + bd_ref[...], 0.0)
    lg = jnp.dot(dd, wo_ref[...], preferred_element_type=_f32) + bo_ref[...]
    colmask = lax.broadcasted_iota(_i32, (G, G), 1) < NCLS
    mx = jnp.max(jnp.where(colmask, lg, -jnp.inf), axis=1, keepdims=True)
    e = jnp.where(colmask, jnp.exp(lg - mx), 0.0)
    out_ref[...] = e / jnp.sum(e, axis=1, keepdims=True)


def _final(s1, cnt, wd, bd, wo, bo):
    full = lambda: (0, 0)
    return pl.pallas_call(
        _final_body,
        grid=(),
        in_specs=[
            pl.BlockSpec((G, F), full),
            pl.BlockSpec((G, G), full),
            pl.BlockSpec((F, F), full),
            pl.BlockSpec((1, F), full),
            pl.BlockSpec((F, G), full),
            pl.BlockSpec((1, G), full),
        ],
        out_specs=pl.BlockSpec((G, G), full),
        out_shape=jax.ShapeDtypeStruct((G, G), _f32),
    )(s1, cnt, wd, bd, wo, bo)


def kernel(x, edge_index, batch,
           gn0_w, gn0_b, gn0_ms, gn1_w, gn1_b, gn1_ms, gn2_w, gn2_b, gn2_ms,
           c1_W1, c1_b1, c1_W2, c1_b2, c1_W3, c1_b3,
           c2_W1, c2_b1, c2_W2, c2_b2, c2_W3, c2_b3,
           c3_W1, c3_b1, c3_W2, c3_b2, c3_W3, c3_b3,
           Wd, bd, Wo, bo):
    src = edge_index[0].astype(_i32)
    dst = edge_index[1].astype(_i32)
    dsts, srcs = lax.sort([dst, src], num_keys=1)
    deg = jnp.zeros((N,), _i32).at[dst].add(1, mode="drop")
    start = jnp.cumsum(deg) - deg
    has_edge = deg > 0
    startc = jnp.minimum(start, E - 1)
    row_idx = jnp.concatenate([startc, jnp.zeros((NP - N,), _i32)])
    mask2d = jnp.concatenate(
        [has_edge.astype(_f32), jnp.zeros((NP - N,), _f32)]).reshape(NP, 1)
    ones2d = jnp.ones((NP, 1), _f32)
    batch2d = jnp.concatenate(
        [batch.astype(_i32), jnp.full((NP - N,), G - 1, _i32)]).reshape(NP, 1)
    dsts2d = dsts.reshape(E, 1)
    x_pad = jnp.concatenate([x, jnp.zeros((NP - N, F), _f32)], axis=0)

    gn = [(gn0_w, gn0_b, gn0_ms), (gn1_w, gn1_b, gn1_ms), (gn2_w, gn2_b, gn2_ms)]
    convs = [(c1_W1, c1_b1, c1_W2, c1_b2, c1_W3, c1_b3),
             (c2_W1, c2_b1, c2_W2, c2_b2, c2_W3, c2_b3),
             (c3_W1, c3_b1, c3_W2, c3_b2, c3_W3, c3_b3)]

    h = x_pad
    m = ones2d
    for i in range(3):
        w, b, ms = gn[i]
        W1, b1, W2, b2, W3, b3 = convs[i]
        w1d = W1[:F] - W1[F:]
        w1b = W1[F:]
        apply_g = i > 0
        s1, s2, cnt = _stats(h, batch2d, m, apply_g)
        p, q = _apply_pq(h, batch2d, m, s1, s2, cnt,
                         w.reshape(1, F), b.reshape(1, F), ms.reshape(1, F),
                         w1d, w1b, apply_g)
        rp, rq = _edge_gather(p, q, dsts, srcs)
        seg = _mlp_segmax(rp, rq, dsts2d,
                          b1.reshape(1, F), b2.reshape(1, F), b3.reshape(1, F),
                          W2.astype(jnp.bfloat16), W3.astype(jnp.bfloat16))
        h = _node_gather(seg, row_idx)
        m = mask2d

    s1, _, cnt = _stats(h, batch2d, m, True)
    wo_pad = jnp.concatenate([Wo, jnp.zeros((F, G - NCLS), _f32)], axis=1)
    bo_pad = jnp.concatenate([bo, jnp.zeros((G - NCLS,), _f32)]).reshape(1, G)
    probs = _final(s1, cnt, Wd, bd.reshape(1, F), wo_pad, bo_pad)
    return probs[:NG, :NCLS]


# speedup vs baseline: 1.0554x; 1.0554x over previous
import functools

import jax
import jax.numpy as jnp
from jax import lax
from jax.experimental import pallas as pl
from jax.experimental.pallas import tpu as pltpu
from jax.experimental.pallas import tpu_sc as plsc

N = 10000
NP = 10240
E = 160000
F = 256
G = 128
NG = 100
NCLS = 10
BN = 2048
BE = 2000
NBE = E // BE

NC = 2
NS = 16
NW = NC * NS
EPW = E // NW
ECH = 40
NBUF = 5
EGRP = EPW // (ECH * NBUF)
NPW = NP // NW
NCH = 80
NITER = NPW // NCH

_f32 = jnp.float32
_i32 = jnp.int32
_u32 = jnp.uint32
FH = F // 2


def _pack_bf16(x):
    return _pack_pair(x.astype(jnp.bfloat16))


def _pack_pair(xb):
    u = lax.bitcast_convert_type(xb, jnp.uint16).astype(_u32)
    return u[:, :FH] | (u[:, FH:] << 16)


def _unpack_bf16(r):
    lo = lax.bitcast_convert_type(r << 16, _f32)
    hi = lax.bitcast_convert_type(r & _u32(0xFFFF0000), _f32)
    return lo, hi


def _g(x, m):
    x = jnp.where(jnp.abs(x) < jnp.inf, x, 0.0)
    return jnp.maximum(x, 0.0) * m


def _selu(x):
    alpha = 1.6732632423543772848170429916717
    scale = 1.0507009873554804934193349852946
    safe = jnp.minimum(x, 0.0)
    return scale * jnp.where(x > 0, x, alpha * (jnp.exp(safe) - 1.0))


def _stats_body(apply_g, x_ref, b_ref, m_ref, s1_ref, s2_ref, cnt_ref):
    pid = pl.program_id(0)

    @pl.when(pid == 0)
    def _():
        s1_ref[...] = jnp.zeros_like(s1_ref)
        s2_ref[...] = jnp.zeros_like(s2_ref)
        cnt_ref[...] = jnp.zeros_like(cnt_ref)

    if apply_g:
        lo, hi = _unpack_bf16(x_ref[...])
        x = _g(jnp.concatenate([lo, hi], axis=1), m_ref[...])
    else:
        x = x_ref[...]
    oh = (lax.broadcasted_iota(_i32, (BN, G), 1) == b_ref[...]).astype(_f32)
    dn = (((0,), (0,)), ((), ()))
    s1_ref[...] += lax.dot_general(oh, x, dn, preferred_element_type=_f32)
    s2_ref[...] += lax.dot_general(oh, x * x, dn, preferred_element_type=_f32)
    cnt_ref[...] += lax.dot_general(oh, jnp.ones((BN, G), _f32), dn,
                                    preferred_element_type=_f32)


def _stats(x, batch2d, mask2d, apply_g):
    grid = NP // BN
    return pl.pallas_call(
        functools.partial(_stats_body, apply_g),
        grid=(grid,),
        in_specs=[
            pl.BlockSpec((BN, FH if apply_g else F), lambda g: (g, 0)),
            pl.BlockSpec((BN, 1), lambda g: (g, 0)),
            pl.BlockSpec((BN, 1), lambda g: (g, 0)),
        ],
        out_specs=[
            pl.BlockSpec((G, F), lambda g: (0, 0)),
            pl.BlockSpec((G, F), lambda g: (0, 0)),
            pl.BlockSpec((G, G), lambda g: (0, 0)),
        ],
        out_shape=[
            jax.ShapeDtypeStruct((G, F), _f32),
            jax.ShapeDtypeStruct((G, F), _f32),
            jax.ShapeDtypeStruct((G, G), _f32),
        ],
    )(x, batch2d, mask2d)


def _pq_body(apply_g, x_ref, b_ref, m_ref, s1_ref, s2_ref, cnt_ref,
             w_ref, bb_ref, ms_ref, w1d_ref, w1b_ref, p_ref, q_ref):
    cnt = jnp.maximum(cnt_ref[:, 0:1], 1.0)
    mean = s1_ref[...] / cnt
    m2 = s2_ref[...] / cnt
    ms = ms_ref[...]
    var = m2 - mean * mean * ms * (2.0 - ms)
    scale = w_ref[...] * lax.rsqrt(var + 1e-5)
    shift = bb_ref[...] - scale * ms * mean
    if apply_g:
        lo, hi = _unpack_bf16(x_ref[...])
        x = _g(jnp.concatenate([lo, hi], axis=1), m_ref[...])
    else:
        x = x_ref[...]
    oh = (lax.broadcasted_iota(_i32, (BN, G), 1) == b_ref[...]).astype(_f32)
    xn = x * jnp.dot(oh, scale, preferred_element_type=_f32) \
        + jnp.dot(oh, shift, preferred_element_type=_f32)
    p_ref[...] = _pack_bf16(jnp.dot(xn, w1d_ref[...],
                                    preferred_element_type=_f32))
    q_ref[...] = _pack_bf16(jnp.dot(xn, w1b_ref[...],
                                    preferred_element_type=_f32))


def _apply_pq(x, batch2d, mask2d, s1, s2, cnt, w, bb, ms, w1d, w1b, apply_g):
    grid = NP // BN
    full = lambda g: (0, 0)
    blk = lambda g: (g, 0)
    return pl.pallas_call(
        functools.partial(_pq_body, apply_g),
        grid=(grid,),
        in_specs=[
            pl.BlockSpec((BN, FH if apply_g else F), blk),
            pl.BlockSpec((BN, 1), blk),
            pl.BlockSpec((BN, 1), blk),
            pl.BlockSpec((G, F), full),
            pl.BlockSpec((G, F), full),
            pl.BlockSpec((G, G), full),
            pl.BlockSpec((1, F), full),
            pl.BlockSpec((1, F), full),
            pl.BlockSpec((1, F), full),
            pl.BlockSpec((F, F), full),
            pl.BlockSpec((F, F), full),
        ],
        out_specs=[
            pl.BlockSpec((BN, FH), blk),
            pl.BlockSpec((BN, FH), blk),
        ],
        out_shape=[
            jax.ShapeDtypeStruct((NP, FH), _u32),
            jax.ShapeDtypeStruct((NP, FH), _u32),
        ],
    )(x, batch2d, mask2d, s1, s2, cnt, w, bb, ms, w1d, w1b)


def _edge_gather_body(p_hbm, q_hbm, d_hbm, s_hbm, rp_hbm, rq_hbm, *scr):
    wid = lax.axis_index("s") * NC + lax.axis_index("c")
    base = wid * EPW
    bufs = [scr[5 * b:5 * b + 5] for b in range(NBUF)]

    def group(i, carry):
        g0 = i * NBUF
        for b, (di, si, pr, qr, sem) in enumerate(bufs):
            off = base + (g0 + b) * ECH

            @pl.when(i > 0)
            def _(pr=pr, qr=qr, off=off, sem=sem):
                pltpu.make_async_copy(pr, rp_hbm.at[pl.ds(off, ECH)], sem).wait()
                pltpu.make_async_copy(qr, rq_hbm.at[pl.ds(off, ECH)], sem).wait()

            pltpu.sync_copy(d_hbm.at[pl.ds(off, ECH)], di)
            pltpu.sync_copy(s_hbm.at[pl.ds(off, ECH)], si)
            pltpu.async_copy(p_hbm.at[di], pr, sem)
            pltpu.async_copy(q_hbm.at[si], qr, sem)
        for b, (di, si, pr, qr, sem) in enumerate(bufs):
            off = base + (g0 + b) * ECH
            pltpu.make_async_copy(p_hbm.at[di], pr, sem).wait()
            pltpu.make_async_copy(q_hbm.at[si], qr, sem).wait()
            pltpu.async_copy(pr, rp_hbm.at[pl.ds(off, ECH)], sem)
            pltpu.async_copy(qr, rq_hbm.at[pl.ds(off, ECH)], sem)
        return carry

    lax.fori_loop(0, EGRP, group, 0)
    for di, si, pr, qr, sem in bufs:
        pltpu.make_async_copy(pr, rp_hbm.at[pl.ds(base, ECH)], sem).wait()
        pltpu.make_async_copy(qr, rq_hbm.at[pl.ds(base, ECH)], sem).wait()


def _edge_gather(p, q, dsts, srcs):
    mesh = plsc.VectorSubcoreMesh(core_axis_name="c", subcore_axis_name="s")
    scratch = []
    for _ in range(NBUF):
        scratch += [
            pltpu.VMEM((ECH,), _i32),
            pltpu.VMEM((ECH,), _i32),
            pltpu.VMEM((ECH, FH), _u32),
            pltpu.VMEM((ECH, FH), _u32),
            pltpu.SemaphoreType.DMA,
        ]
    fn = pl.kernel(
        _edge_gather_body,
        out_type=[
            jax.ShapeDtypeStruct((E, FH), _u32),
            jax.ShapeDtypeStruct((E, FH), _u32),
        ],
        mesh=mesh,
        scratch_types=scratch,
    )
    return fn(p, q, dsts, srcs)


def _mlp_segmax_body(rp_ref, rq_ref, d_ref, b1_ref, b2_ref, b3_ref,
                     w2_ref, w3_ref, out_ref, cd_ref, cv_ref):
    pid = pl.program_id(0)

    @pl.when(pid == 0)
    def _():
        cd_ref[...] = jnp.full(cd_ref.shape, -1, _i32)
        cv_ref[...] = jnp.full(cv_ref.shape, -jnp.inf, jnp.bfloat16)

    plo, phi = _unpack_bf16(rp_ref[...])
    qlo, qhi = _unpack_bf16(rq_ref[...])
    b1 = b1_ref[...]
    h = _selu(jnp.concatenate(
        [plo + qlo + b1[:, :FH], phi + qhi + b1[:, FH:]], axis=1))
    h = _selu(jnp.dot(h.astype(jnp.bfloat16), w2_ref[...],
                      preferred_element_type=_f32) + b2_ref[...])
    h = jnp.dot(h.astype(jnp.bfloat16), w3_ref[...],
                preferred_element_type=_f32) + b3_ref[...]
    hb = h.astype(jnp.bfloat16)
    d = d_ref[...]
    s = 1
    while s < BE:
        hs = jnp.concatenate(
            [hb[s:], jnp.zeros((s, F), jnp.bfloat16)], axis=0)
        ds = jnp.concatenate([d[s:], jnp.full((s, 1), -1, _i32)], axis=0)
        hb = jnp.where(ds == d, jnp.maximum(hb, hs), hb)
        s *= 2
    cd = cd_ref[0:1, 0:1]
    cv = cv_ref[0:1, :]
    hb = jnp.where(d == cd, jnp.maximum(hb, cv), hb)
    out_ref[...] = _pack_pair(hb)
    cd_ref[0:1, 0:1] = d[0:1, :]
    cv_ref[0:1, :] = hb[0:1, :]


def _mlp_segmax(rp, rq, dsts2d, b1, b2, b3, w2, w3):
    desc = lambda g: (NBE - 1 - g, 0)
    full = lambda g: (0, 0)
    return pl.pallas_call(
        _mlp_segmax_body,
        grid=(NBE,),
        in_specs=[
            pl.BlockSpec((BE, FH), desc),
            pl.BlockSpec((BE, FH), desc),
            pl.BlockSpec((BE, 1), desc),
            pl.BlockSpec((1, F), full),
            pl.BlockSpec((1, F), full),
            pl.BlockSpec((1, F), full),
            pl.BlockSpec((F, F), full),
            pl.BlockSpec((F, F), full),
        ],
        out_specs=pl.BlockSpec((BE, FH), desc),
        out_shape=jax.ShapeDtypeStruct((E, FH), _u32),
        scratch_shapes=[
            pltpu.VMEM((8, 128), _i32),
            pltpu.VMEM((8, F), jnp.bfloat16),
        ],
    )(rp, rq, dsts2d, b1, b2, b3, w2, w3)


def _node_gather_body(s_hbm, idx_hbm, out_hbm, ix_v, rows_v, sem):
    wid = lax.axis_index("s") * NC + lax.axis_index("c")
    base = wid * NPW

    def step(j, carry):
        off = base + j * NCH
        pltpu.sync_copy(idx_hbm.at[pl.ds(off, NCH)], ix_v)
        pltpu.async_copy(s_hbm.at[ix_v], rows_v, sem).wait()
        pltpu.sync_copy(rows_v, out_hbm.at[pl.ds(off, NCH)])
        return carry

    lax.fori_loop(0, NITER, step, 0)


def _node_gather(seg, row_idx):
    mesh = plsc.VectorSubcoreMesh(core_axis_name="c", subcore_axis_name="s")
    fn = pl.kernel(
        _node_gather_body,
        out_type=jax.ShapeDtypeStruct((NP, FH), _u32),
        mesh=mesh,
        scratch_types=[
            pltpu.VMEM((NCH,), _i32),
            pltpu.VMEM((NCH, FH), _u32),
            pltpu.SemaphoreType.DMA,
        ],
    )
    return fn(seg, row_idx)


def _final_body(s1_ref, cnt_ref, wd_ref, bd_ref, wo_ref, bo_ref, out_ref):
    cnt = jnp.maximum(cnt_ref[:, 0:1], 1.0)
    pooled = s1_ref[...] / cnt
    dd = jnp.maximum(
        jnp.dot(pooled, wd_ref[...], preferred_element_type=_f32)
        + bd_ref[...], 0.0)
    lg = jnp.dot(dd, wo_ref[...], preferred_element_type=_f32) + bo_ref[...]
    colmask = lax.broadcasted_iota(_i32, (G, G), 1) < NCLS
    mx = jnp.max(jnp.where(colmask, lg, -jnp.inf), axis=1, keepdims=True)
    e = jnp.where(colmask, jnp.exp(lg - mx), 0.0)
    out_ref[...] = e / jnp.sum(e, axis=1, keepdims=True)


def _final(s1, cnt, wd, bd, wo, bo):
    full = lambda: (0, 0)
    return pl.pallas_call(
        _final_body,
        grid=(),
        in_specs=[
            pl.BlockSpec((G, F), full),
            pl.BlockSpec((G, G), full),
            pl.BlockSpec((F, F), full),
            pl.BlockSpec((1, F), full),
            pl.BlockSpec((F, G), full),
            pl.BlockSpec((1, G), full),
        ],
        out_specs=pl.BlockSpec((G, G), full),
        out_shape=jax.ShapeDtypeStruct((G, G), _f32),
    )(s1, cnt, wd, bd, wo, bo)


def kernel(x, edge_index, batch,
           gn0_w, gn0_b, gn0_ms, gn1_w, gn1_b, gn1_ms, gn2_w, gn2_b, gn2_ms,
           c1_W1, c1_b1, c1_W2, c1_b2, c1_W3, c1_b3,
           c2_W1, c2_b1, c2_W2, c2_b2, c2_W3, c2_b3,
           c3_W1, c3_b1, c3_W2, c3_b2, c3_W3, c3_b3,
           Wd, bd, Wo, bo):
    src = edge_index[0].astype(_i32)
    dst = edge_index[1].astype(_i32)
    dsts, srcs = lax.sort([dst, src], num_keys=1)
    deg = jnp.zeros((N,), _i32).at[dst].add(1, mode="drop")
    start = jnp.cumsum(deg) - deg
    has_edge = deg > 0
    startc = jnp.minimum(start, E - 1)
    row_idx = jnp.concatenate([startc, jnp.zeros((NP - N,), _i32)])
    mask2d = jnp.concatenate(
        [has_edge.astype(_f32), jnp.zeros((NP - N,), _f32)]).reshape(NP, 1)
    ones2d = jnp.ones((NP, 1), _f32)
    batch2d = jnp.concatenate(
        [batch.astype(_i32), jnp.full((NP - N,), G - 1, _i32)]).reshape(NP, 1)
    dsts2d = dsts.reshape(E, 1)
    x_pad = jnp.concatenate([x, jnp.zeros((NP - N, F), _f32)], axis=0)

    gn = [(gn0_w, gn0_b, gn0_ms), (gn1_w, gn1_b, gn1_ms), (gn2_w, gn2_b, gn2_ms)]
    convs = [(c1_W1, c1_b1, c1_W2, c1_b2, c1_W3, c1_b3),
             (c2_W1, c2_b1, c2_W2, c2_b2, c2_W3, c2_b3),
             (c3_W1, c3_b1, c3_W2, c3_b2, c3_W3, c3_b3)]

    h = x_pad
    m = ones2d
    for i in range(3):
        w, b, ms = gn[i]
        W1, b1, W2, b2, W3, b3 = convs[i]
        w1d = W1[:F] - W1[F:]
        w1b = W1[F:]
        apply_g = i > 0
        s1, s2, cnt = _stats(h, batch2d, m, apply_g)
        p, q = _apply_pq(h, batch2d, m, s1, s2, cnt,
                         w.reshape(1, F), b.reshape(1, F), ms.reshape(1, F),
                         w1d, w1b, apply_g)
        rp, rq = _edge_gather(p, q, dsts, srcs)
        seg = _mlp_segmax(rp, rq, dsts2d,
                          b1.reshape(1, F), b2.reshape(1, F), b3.reshape(1, F),
                          W2.astype(jnp.bfloat16), W3.astype(jnp.bfloat16))
        h = _node_gather(seg, row_idx)
        m = mask2d

    s1, _, cnt = _stats(h, batch2d, m, True)
    wo_pad = jnp.concatenate([Wo, jnp.zeros((F, G - NCLS), _f32)], axis=1)
    bo_pad = jnp.concatenate([bo, jnp.zeros((G - NCLS,), _f32)]).reshape(1, G)
    probs = _final(s1, cnt, Wd, bd.reshape(1, F), wo_pad, bo_pad)
    return probs[:NG, :NCLS]
